# baseline (speedup 1.0000x reference)
import functools

import jax
import jax.numpy as jnp
from jax import lax
from jax.experimental import pallas as pl
from jax.experimental.pallas import tpu as pltpu
from jax.experimental.pallas import tpu_sc as plsc

HI = jax.lax.Precision.HIGHEST

B, S, NN, IN = 4, 12, 1000, 16
E = 16000
GH, HEADS, LH = 32, 4, 64

C = 32
EP = E
NW = 32
G = B * S
N = G * NN
TW = C * NN


def _edge_stage_body(XLf, XRf, ATT, EPK, OUT, xlh, xrh, outtab, ebuf, epk,
                     eself, attv, *, T):
    cid = lax.axis_index("c")
    sid = lax.axis_index("s")
    wid = sid * 2 + cid
    rounds = (T + NW - 1) // NW

    zeros16 = jnp.zeros((16,), jnp.float32)
    iota16 = lax.iota(jnp.int32, 16)
    UROW = C * NN

    pltpu.sync_copy(EPK, epk)

    def task_body(t):
        pltpu.sync_copy(XLf.at[t], xlh)
        pltpu.sync_copy(XRf.at[t], xrh.at[pl.ds(0, TW)])
        pltpu.sync_copy(ATT.at[t], attv)

        @plsc.parallel_loop(0, TW // 64)
        def _zo(i):
            outtab[pl.ds(i * 64, 16)] = zeros16
            outtab[pl.ds(i * 64 + 16, 16)] = zeros16
            outtab[pl.ds(i * 64 + 32, 16)] = zeros16
            outtab[pl.ds(i * 64 + 48, 16)] = zeros16

        def bmax(i, mm):
            v = jnp.abs(xlh[pl.ds(i * 16, 16)])
            u = jnp.abs(xrh[pl.ds(i * 16, 16)])
            return jnp.maximum(mm, jnp.maximum(v, u))

        gm = jnp.max(lax.fori_loop(0, TW // 16, bmax, zeros16))
        alo0 = attv[pl.ds(0, 16)]
        ahi0 = attv[pl.ds(16, 16)]
        satt = jnp.sum(jnp.abs(alo0)) + jnp.sum(jnp.abs(ahi0))
        m = 3.5 * satt * gm

        urow = xlh.at[pl.ds(UROW, NN)]

        def p1(g, carry):
            alo = attv[pl.ds(0, 16)]
            ahi = attv[pl.ds(16, 16)]
            w0 = epk[pl.ds(g * 32, 16)]
            w1 = epk[pl.ds(g * 32 + 16, 16)]
            s0 = jnp.right_shift(w0, 11)
            d0 = jnp.bitwise_and(w0, 2047)
            s1 = jnp.right_shift(w1, 11)
            d1 = jnp.bitwise_and(w1, 2047)
            acc = [zeros16, zeros16, zeros16, zeros16]
            for c in range(C):
                xls = xlh.at[pl.ds(c * NN, NN)]
                xrs = xrh.at[pl.ds(c * NN, NN)]
                a0 = plsc.load_gather(xls, [s0])
                b0 = plsc.load_gather(xrs, [d0])
                a1 = plsc.load_gather(xls, [s1])
                b1 = plsc.load_gather(xrs, [d1])
                z0 = jnp.abs(a0 + b0)
                z1 = jnp.abs(a1 + b1)
                asc = alo[c] if c < 16 else ahi[c - 16]
                acc[2 * (c % 2)] = acc[2 * (c % 2)] + z0 * asc
                acc[2 * (c % 2) + 1] = acc[2 * (c % 2) + 1] + z1 * asc
            u0 = plsc.load_gather(urow, [s0])
            u1 = plsc.load_gather(urow, [s1])
            ebuf[pl.ds(g * 32, 16)] = jnp.exp(u0 + (acc[0] + acc[2]) - m)
            ebuf[pl.ds(g * 32 + 16, 16)] = jnp.exp(u1 + (acc[1] + acc[3]) - m)
            return carry

        lax.fori_loop(0, EP // 32, p1, 0)

        @plsc.parallel_loop(0, 63)
        def _ps(k):
            alo = attv[pl.ds(0, 16)]
            ahi = attv[pl.ds(16, 16)]
            acc = [zeros16, zeros16]
            for c in range(C):
                xv = xlh[pl.ds(c * NN + k * 16, 16)]
                yv = xrh[pl.ds(c * NN + k * 16, 16)]
                z = jnp.abs(xv + yv)
                asc = alo[c] if c < 16 else ahi[c - 16]
                acc[c % 2] = acc[c % 2] + z * asc
            uv = xlh[pl.ds(UROW + k * 16, 16)]
            eS = jnp.exp(uv + (acc[0] + acc[1]) - m)
            eself[pl.ds(k * 16, 16)] = eS
            xrh[pl.ds(k * 16, 16)] = eS

        @plsc.parallel_loop(0, EP // 16)
        def _p2(g):
            w = epk[pl.ds(g * 16, 16)]
            d16 = jnp.bitwise_and(w, 2047)
            e16 = ebuf[pl.ds(g * 16, 16)]
            plsc.addupdate_scatter(xrh, [d16], e16)

        @plsc.parallel_loop(0, EP // 16)
        def _p3(g):
            w = epk[pl.ds(g * 16, 16)]
            s16 = jnp.right_shift(w, 11)
            d16 = jnp.bitwise_and(w, 2047)
            ss = plsc.load_gather(xrh, [d16])
            a16 = ebuf[pl.ds(g * 16, 16)] / (ss + 1e-16)
            for c in range(C):
                xls = xlh.at[pl.ds(c * NN, NN)]
                v = plsc.load_gather(xls, [s16]) * a16
                plsc.addupdate_scatter(outtab.at[pl.ds(c * NN, NN)], [d16], v)

        tmask = iota16 < 8

        @plsc.parallel_loop(0, 62)
        def _pm(k):
            eS = eself[pl.ds(k * 16, 16)]
            ss = xrh[pl.ds(k * 16, 16)]
            aS = eS / (ss + 1e-16)
            for c in range(C):
                o = outtab[pl.ds(c * NN + k * 16, 16)]
                xv = xlh[pl.ds(c * NN + k * 16, 16)]
                outtab[pl.ds(c * NN + k * 16, 16)] = o + aS * xv

        eS = eself[pl.ds(992, 16)]
        ss = xrh[pl.ds(992, 16)]
        aS = eS / (ss + 1e-16)
        for c in range(C):
            o = outtab[pl.ds(c * NN + 992, 16)]
            xv = xlh[pl.ds(c * NN + 992, 16)]
            outtab[pl.ds(c * NN + 992, 16)] = jnp.where(tmask, o + aS * xv, o)

        pltpu.sync_copy(outtab.at[pl.ds(0, TW)], OUT.at[t])

    def round_body(r, carry):
        t = r * NW + wid
        if T % NW == 0:
            task_body(t)
        else:
            @pl.when(t < T)
            def _():
                task_body(t)
        return carry

    lax.fori_loop(0, rounds, round_body, 0)


def _edge_stage(T, XLf, XRf, ATT, EPK):
    mesh = plsc.VectorSubcoreMesh(core_axis_name="c", subcore_axis_name="s")
    f = pl.kernel(
        functools.partial(_edge_stage_body, T=T),
        out_type=jax.ShapeDtypeStruct((T, TW), jnp.float32),
        mesh=mesh,
        compiler_params=pltpu.CompilerParams(needs_layout_passes=False),
        scratch_types=[
            pltpu.VMEM(((C + 1) * NN,), jnp.float32),
            pltpu.VMEM((TW + 8,), jnp.float32),
            pltpu.VMEM((TW + 8,), jnp.float32),
            pltpu.VMEM((EP,), jnp.float32),
            pltpu.VMEM((EP,), jnp.int32),
            pltpu.VMEM((1008,), jnp.float32),
            pltpu.VMEM((C,), jnp.float32),
        ],
    )
    return f(XLf, XRf, ATT, EPK)


_CMAJ = (((1,), (1,)), ((), ()))


def _proj1_body(x_ref, wl_ref, bl_ref, wr_ref, br_ref, att_ref,
                xl_ref, xr_ref):
    xg = x_ref[...]
    xl = lax.dot_general(wl_ref[...], xg, _CMAJ, precision=HI,
                         preferred_element_type=jnp.float32) + bl_ref[...]
    xr = lax.dot_general(wr_ref[...], xg, _CMAJ, precision=HI,
                         preferred_element_type=jnp.float32) + br_ref[...]
    att = att_ref[...]
    for h in range(HEADS):
        xlh = xl[h * C:(h + 1) * C, :]
        u = 0.6 * jnp.sum(xlh * att[h][:, None], axis=0, keepdims=True)
        xl_ref[0, h] = jnp.concatenate([xlh, u], axis=0)
        xr_ref[0, h] = xr[h * C:(h + 1) * C, :]


def _proj1(xf, W1l, b1l, W1r, b1r, att1):
    wspec = lambda shape: pl.BlockSpec(shape, lambda g: (0,) * len(shape))
    return pl.pallas_call(
        _proj1_body,
        grid=(G,),
        in_specs=[
            pl.BlockSpec((NN, IN), lambda g: (g, 0)),
            wspec((HEADS * GH, IN)), wspec((HEADS * GH, 1)),
            wspec((HEADS * GH, IN)), wspec((HEADS * GH, 1)),
            wspec((HEADS, C)),
        ],
        out_specs=[
            pl.BlockSpec((1, HEADS, C + 1, NN), lambda g: (g, 0, 0, 0)),
            pl.BlockSpec((1, HEADS, C, NN), lambda g: (g, 0, 0, 0)),
        ],
        out_shape=[jax.ShapeDtypeStruct((G, HEADS, C + 1, NN), jnp.float32),
                   jax.ShapeDtypeStruct((G, HEADS, C, NN), jnp.float32)],
    )(xf, W1l, b1l.reshape(-1, 1), W1r, b1r.reshape(-1, 1), att1)


def _mid_body(o1_ref, bias1_ref, g1_ref, be1_ref, w2l_ref, b2l_ref,
              w2r_ref, b2r_ref, att2_ref, xl2_ref, xr2_ref):
    h = jnp.concatenate([o1_ref[0, i] for i in range(HEADS)], axis=0)
    h = h + bias1_ref[...]
    mu = jnp.mean(h, axis=0, keepdims=True)
    var = jnp.mean((h - mu) * (h - mu), axis=0, keepdims=True)
    h = (h - mu) / jnp.sqrt(var + 1e-5) * g1_ref[...] + be1_ref[...]
    h = jnp.maximum(h, 0.0)
    xl2 = jnp.dot(w2l_ref[...], h, precision=HI,
                  preferred_element_type=jnp.float32) + b2l_ref[...]
    u2 = 0.6 * jnp.sum(xl2 * att2_ref[...].reshape(C, 1), axis=0,
                       keepdims=True)
    xl2_ref[0] = jnp.concatenate([xl2, u2], axis=0)
    xr2_ref[0] = jnp.dot(w2r_ref[...], h, precision=HI,
                         preferred_element_type=jnp.float32) + b2r_ref[...]


def _mid(out1, bias1, g1, be1, W2l, b2l, W2r, b2r, att2):
    wspec = lambda shape: pl.BlockSpec(shape, lambda g: (0,) * len(shape))
    return pl.pallas_call(
        _mid_body,
        grid=(G,),
        in_specs=[
            pl.BlockSpec((1, HEADS, C, NN), lambda g: (g, 0, 0, 0)),
            wspec((HEADS * GH, 1)), wspec((HEADS * GH, 1)), wspec((HEADS * GH, 1)),
            wspec((GH, HEADS * GH)), wspec((GH, 1)),
            wspec((GH, HEADS * GH)), wspec((GH, 1)),
            wspec((1, C)),
        ],
        out_specs=[
            pl.BlockSpec((1, C + 1, NN), lambda g: (g, 0, 0)),
            pl.BlockSpec((1, C, NN), lambda g: (g, 0, 0)),
        ],
        out_shape=[jax.ShapeDtypeStruct((G, C + 1, NN), jnp.float32),
                   jax.ShapeDtypeStruct((G, C, NN), jnp.float32)],
    )(out1, bias1.reshape(-1, 1), g1.reshape(-1, 1), be1.reshape(-1, 1),
      W2l, b2l.reshape(-1, 1), W2r, b2r.reshape(-1, 1), att2)


ROW_TILE = NN


def _lstm_body(o2_ref, bias2_ref, g2_ref, be2_ref, Wih0_ref, Whh0_ref, b0_ref,
               Wih1_ref, Whh1_ref, b1_ref, Wfc_ref, bfc_ref, out_ref):
    R = ROW_TILE
    wih0 = Wih0_ref[...]
    whh0 = Whh0_ref[...]
    b0 = b0_ref[...]
    wih1 = Wih1_ref[...]
    whh1 = Whh1_ref[...]
    b1 = b1_ref[...]
    g2 = g2_ref[...]
    be2 = be2_ref[...]
    bias2 = bias2_ref[...]

    def cell(xt, h, c, wih, whh, b):
        gates = (jnp.dot(wih, xt, precision=HI, preferred_element_type=jnp.float32)
                 + jnp.dot(whh, h, precision=HI, preferred_element_type=jnp.float32) + b)
        i = jax.nn.sigmoid(gates[0 * LH:1 * LH, :])
        f = jax.nn.sigmoid(gates[1 * LH:2 * LH, :])
        g = jnp.tanh(gates[2 * LH:3 * LH, :])
        o = jax.nn.sigmoid(gates[3 * LH:4 * LH, :])
        c = f * c + i * g
        h = o * jnp.tanh(c)
        return h, c

    h0 = jnp.zeros((LH, R), jnp.float32)
    c0 = jnp.zeros((LH, R), jnp.float32)
    h1 = jnp.zeros((LH, R), jnp.float32)
    c1 = jnp.zeros((LH, R), jnp.float32)
    for t in range(S):
        xt = o2_ref[0, t] + bias2
        mu = jnp.mean(xt, axis=0, keepdims=True)
        var = jnp.mean((xt - mu) * (xt - mu), axis=0, keepdims=True)
        xt = (xt - mu) / jnp.sqrt(var + 1e-5) * g2 + be2
        h0, c0 = cell(xt, h0, c0, wih0, whh0, b0)
        h1, c1 = cell(h0, h1, c1, wih1, whh1, b1)
    pred = jnp.sum(h1 * Wfc_ref[...], axis=0, keepdims=True)
    out_ref[0] = pred + bfc_ref[0, 0]


def _lstm_fc(out2, bias2, g2, be2, Wih0, Whh0, bih0, bhh0,
             Wih1, Whh1, bih1, bhh1, Wfc, bfc):
    o2v = out2.reshape(B, S, C, NN)
    b0 = (bih0 + bhh0).reshape(4 * LH, 1)
    b1 = (bih1 + bhh1).reshape(4 * LH, 1)
    wspec = lambda shape: pl.BlockSpec(shape, lambda b: (0,) * len(shape))
    grid = (B,)
    return pl.pallas_call(
        _lstm_body,
        grid=grid,
        in_specs=[
            pl.BlockSpec((1, S, C, NN), lambda b: (b, 0, 0, 0)),
            wspec((C, 1)), wspec((C, 1)), wspec((C, 1)),
            wspec((4 * LH, GH)), wspec((4 * LH, LH)), wspec((4 * LH, 1)),
            wspec((4 * LH, LH)), wspec((4 * LH, LH)), wspec((4 * LH, 1)),
            wspec((LH, 1)), wspec((1, 1)),
        ],
        out_specs=pl.BlockSpec((1, 1, NN), lambda b: (b, 0, 0)),
        out_shape=jax.ShapeDtypeStruct((B, 1, NN), jnp.float32),
    )(o2v, bias2.reshape(-1, 1), g2.reshape(-1, 1), be2.reshape(-1, 1),
      Wih0, Whh0, b0, Wih1, Whh1, b1, Wfc.reshape(-1, 1),
      bfc.reshape(1, 1))


def kernel(x, edge_index, W1l, b1l, W1r, b1r, att1, bias1, g1, be1,
           W2l, b2l, W2r, b2r, att2, bias2, g2, be2,
           Wih0, Whh0, bih0, bhh0, Wih1, Whh1, bih1, bhh1, Wfc, bfc):
    xf = x.reshape(N, IN)

    epk = (edge_index[0].astype(jnp.int32) * 2048
           + edge_index[1].astype(jnp.int32))

    XL1, XR1 = _proj1(xf, W1l, b1l, W1r, b1r, att1)
    T1 = G * HEADS
    ATT1 = 0.4 * jnp.tile(att1.astype(jnp.float32), (G, 1))
    out1 = _edge_stage(T1, XL1.reshape(T1, (C + 1) * NN),
                       XR1.reshape(T1, TW), ATT1, epk)

    XL2, XR2 = _mid(out1.reshape(G, HEADS, C, NN), bias1, g1, be1,
                    W2l, b2l, W2r, b2r, att2)

    ATT2 = 0.4 * jnp.tile(att2.astype(jnp.float32), (G, 1))
    out2 = _edge_stage(G, XL2.reshape(G, (C + 1) * NN),
                       XR2.reshape(G, TW), ATT2, epk)

    pred = _lstm_fc(out2.reshape(G, C, NN), bias2, g2, be2,
                     Wih0, Whh0, bih0, bhh0, Wih1, Whh1, bih1, bhh1, Wfc, bfc)
    return pred.reshape(B, NN)
